# speedup vs baseline: 2.4213x; 2.4213x over previous
import functools

import jax
import jax.numpy as jnp
from jax import lax
from jax.experimental import pallas as pl
from jax.experimental.pallas import tpu as pltpu
from jax.experimental.pallas import tpu_sc as plsc

NUM_POSES = 1000
EMBED_DIM = 128
BATCH = 16384

NC = 2
NS = 16
NW = NC * NS
B_PER_W = BATCH // NW


def _make_gather():
    mesh = plsc.VectorSubcoreMesh(core_axis_name="c", subcore_axis_name="s")

    @functools.partial(
        pl.kernel,
        mesh=mesh,
        out_type=jax.ShapeDtypeStruct((BATCH, EMBED_DIM), jnp.float32),
        scratch_types=[
            pltpu.VMEM((B_PER_W,), jnp.int32),
            pltpu.VMEM((B_PER_W, EMBED_DIM), jnp.float32),
            pltpu.SemaphoreType.DMA,
        ],
    )
    def gather_kernel(x_hbm, table_hbm, out_hbm, idx_v, rows_v, sem):
        wid = lax.axis_index("s") * NC + lax.axis_index("c")
        base = wid * B_PER_W
        pltpu.sync_copy(x_hbm.at[pl.ds(base, B_PER_W)], idx_v)
        pltpu.async_copy(table_hbm.at[idx_v], rows_v, sem).wait()
        pltpu.sync_copy(rows_v, out_hbm.at[pl.ds(base, B_PER_W)])

    return gather_kernel


_gather = jax.jit(_make_gather())


def kernel(x, table):
    return _gather(x, table)


# speedup vs baseline: 2.7641x; 1.1416x over previous
import functools

import jax
import jax.numpy as jnp
from jax import lax
from jax.experimental import pallas as pl
from jax.experimental.pallas import tpu as pltpu
from jax.experimental.pallas import tpu_sc as plsc

NUM_POSES = 1000
EMBED_DIM = 128
BATCH = 16384

NC = 2
NS = 16
NW = NC * NS
B_PER_W = BATCH // NW
CHUNK = 128
NCHUNK = B_PER_W // CHUNK
STAGE_TILES = 5
STAGE_ROWS = NUM_POSES // STAGE_TILES


def _make_gather():
    mesh = plsc.VectorSubcoreMesh(core_axis_name="c", subcore_axis_name="s")

    @functools.partial(
        pl.kernel,
        mesh=mesh,
        out_type=jax.ShapeDtypeStruct((BATCH, EMBED_DIM), jnp.float32),
        scratch_types=[
            pltpu.VMEM_SHARED((NUM_POSES, EMBED_DIM), jnp.float32),
            pltpu.VMEM((B_PER_W,), jnp.int32),
            pltpu.VMEM((CHUNK, EMBED_DIM), jnp.float32),
            pltpu.VMEM((CHUNK, EMBED_DIM), jnp.float32),
            pltpu.SemaphoreType.DMA,
            pltpu.SemaphoreType.DMA,
        ],
    )
    def gather_kernel(x_hbm, table_hbm, out_hbm, tab_s, idx_v, buf0, buf1,
                      gsem, wsem):
        sid = lax.axis_index("s")
        wid = sid * NC + lax.axis_index("c")
        base = wid * B_PER_W

        @pl.when(sid < STAGE_TILES)
        def _stage():
            r0 = sid * STAGE_ROWS
            pltpu.sync_copy(table_hbm.at[pl.ds(r0, STAGE_ROWS)],
                            tab_s.at[pl.ds(r0, STAGE_ROWS)])

        pltpu.sync_copy(x_hbm.at[pl.ds(base, B_PER_W)], idx_v)
        plsc.subcore_barrier()

        bufs = (buf0, buf1)
        gathers = []
        writes = []
        for c in range(NCHUNK):
            gathers.append(pltpu.async_copy(
                tab_s.at[idx_v.at[pl.ds(c * CHUNK, CHUNK)]],
                bufs[c % 2], gsem))
            if c >= 2:
                writes[c - 2].wait()
            if c >= 1:
                gathers[c - 1].wait()
                writes.append(pltpu.async_copy(
                    bufs[(c - 1) % 2],
                    out_hbm.at[pl.ds(base + (c - 1) * CHUNK, CHUNK)], wsem))
        gathers[NCHUNK - 1].wait()
        writes.append(pltpu.async_copy(
            bufs[(NCHUNK - 1) % 2],
            out_hbm.at[pl.ds(base + (NCHUNK - 1) * CHUNK, CHUNK)], wsem))
        writes[NCHUNK - 2].wait()
        writes[NCHUNK - 1].wait()

    return gather_kernel


_gather = jax.jit(_make_gather())


def kernel(x, table):
    return _gather(x, table)
